# Initial kernel scaffold; baseline (speedup 1.0000x reference)
import jax
import jax.numpy as jnp
from jax.experimental import pallas as pl


def kernel(node_fea, edge_fea, idx1, idx2, idx3, params):
    raise NotImplementedError("write your pallas kernel here")


# speedup vs baseline: 13.1749x; 13.1749x over previous
import functools

import jax
import jax.numpy as jnp
from jax import lax
from jax.experimental import pallas as pl
from jax.experimental.pallas import tpu as pltpu
from jax.experimental.pallas import tpu_sc as plsc

F32 = jnp.float32
I32 = jnp.int32
HI = jax.lax.Precision.HIGHEST

_NC = 2
_NS = 16
_NW = _NC * _NS

_EDGE_CHUNK = 2000
_EDGE_BLK = 1600

_SC_PARAMS = pltpu.CompilerParams(use_tc_tiling_on_sc=False)


def _sc_mesh():
    return plsc.VectorSubcoreMesh(core_axis_name="c", subcore_axis_name="s",
                                  num_cores=_NC, num_subcores=_NS)


def _wid():
    return lax.axis_index("s") * _NC + lax.axis_index("c")


def _lrelu(x):
    return jnp.where(x >= 0, x, 0.2 * x)


def _bd(w):
    return jnp.kron(jnp.eye(8, dtype=F32), w.T)


def _tile8(b):
    return jnp.tile(b, 8)[None, :]


def _sc_emb_gather(table, idx, n_pad, d):
    rpw = n_pad // _NW

    @functools.partial(
        pl.kernel,
        out_type=jax.ShapeDtypeStruct((n_pad, d), F32),
        mesh=_sc_mesh(),
        compiler_params=_SC_PARAMS,
        scratch_types=[
            pltpu.VMEM((rpw,), I32),
            pltpu.VMEM((rpw, d), F32),
            pltpu.SemaphoreType.DMA,
        ],
    )
    def k(table_hbm, idx_hbm, out_hbm, idx_v, rows_v, sem):
        base = _wid() * rpw
        pltpu.sync_copy(idx_hbm.at[pl.ds(base, rpw)], idx_v)
        pltpu.async_copy(table_hbm.at[idx_v], rows_v, sem).wait()
        pltpu.sync_copy(rows_v, out_hbm.at[pl.ds(base, rpw)])

    return k(table, idx)


def _sc_edge_gather(a1, a2, idx1, idx2, e_edges):
    chunk = _EDGE_CHUNK
    epw = e_edges // _NW
    nch = epw // chunk

    @functools.partial(
        pl.kernel,
        out_type=(
            jax.ShapeDtypeStruct((e_edges, 16), F32),
            jax.ShapeDtypeStruct((e_edges, 16), F32),
        ),
        mesh=_sc_mesh(),
        compiler_params=_SC_PARAMS,
        scratch_types=[
            pltpu.VMEM((chunk,), I32),
            pltpu.VMEM((chunk,), I32),
            pltpu.VMEM((chunk, 16), F32),
            pltpu.VMEM((chunk, 16), F32),
            pltpu.SemaphoreType.DMA,
        ],
    )
    def k(a1_hbm, a2_hbm, i1_hbm, i2_hbm, g1_hbm, g2_hbm, i1v, i2v, r1v, r2v, sem):
        base0 = _wid() * epw
        for c in range(nch):
            b = base0 + c * chunk
            pltpu.sync_copy(i1_hbm.at[pl.ds(b, chunk)], i1v)
            pltpu.sync_copy(i2_hbm.at[pl.ds(b, chunk)], i2v)
            d1 = pltpu.async_copy(a1_hbm.at[i1v], r1v, sem)
            d2 = pltpu.async_copy(a2_hbm.at[i2v], r2v, sem)
            d1.wait()
            d2.wait()
            pltpu.sync_copy(r1v, g1_hbm.at[pl.ds(b, chunk)])
            pltpu.sync_copy(r2v, g2_hbm.at[pl.ds(b, chunk)])

    return k(a1, a2, idx1, idx2)


def _sc_seg_sum(vals, widths, idx, zeros_map, n_rows, n_items, chunk):
    nv = len(vals)
    ipc = n_items // _NC
    ips = ipc // _NS
    nch = ips // chunk

    out_type = tuple(
        jax.ShapeDtypeStruct((_NC * n_rows, w), F32) for w in widths
    )
    scratch = (
        [pltpu.VMEM((chunk,), I32)]
        + [pltpu.VMEM((chunk, w), F32) for w in widths]
        + [pltpu.VMEM_SHARED((n_rows, w), F32) for w in widths]
    )
    zero_keys = sorted(zeros_map)
    zeros_in = [zeros_map[w] for w in zero_keys]
    nz = len(zeros_in)

    @functools.partial(
        pl.kernel,
        out_type=out_type if nv > 1 else out_type[0],
        mesh=_sc_mesh(),
        compiler_params=_SC_PARAMS,
        scratch_types=scratch,
    )
    def k(*refs):
        idx_hbm = refs[0]
        zero_hbms = {w: refs[1 + i] for i, w in enumerate(zero_keys)}
        val_hbms = refs[1 + nz : 1 + nz + nv]
        out_hbms = refs[1 + nz + nv : 1 + nz + 2 * nv]
        iv = refs[1 + nz + 2 * nv]
        vvs = refs[2 + nz + 2 * nv : 2 + nz + 3 * nv]
        shs = refs[2 + nz + 3 * nv : 2 + nz + 4 * nv]

        cid = lax.axis_index("c")
        sid = lax.axis_index("s")

        @pl.when(sid == 0)
        def _():
            for w, sh in zip(widths, shs):
                pltpu.sync_copy(zero_hbms[w], sh)

        plsc.subcore_barrier()
        base0 = cid * ipc + sid * ips
        for c in range(nch):
            b = base0 + c * chunk
            pltpu.sync_copy(idx_hbm.at[pl.ds(b, chunk)], iv)
            for vhbm, vv, sh in zip(val_hbms, vvs, shs):
                pltpu.sync_copy(vhbm.at[pl.ds(b, chunk)], vv)
                pltpu.sync_copy(vv, sh.at[iv], add=True)
        plsc.subcore_barrier()

        @pl.when(sid == 0)
        def _():
            for sh, ohbm in zip(shs, out_hbms):
                pltpu.sync_copy(sh, ohbm.at[pl.ds(cid * n_rows, n_rows)])

    outs = k(idx, *zeros_in, *vals)
    outs = outs if nv > 1 else (outs,)
    return [o.reshape(_NC, n_rows, w) for o, w in zip(outs, widths)]


def _sc_seg_count(ones_c16, idx, zeros_16, n_rows, n_items, chunk):
    ipc = n_items // _NC
    ips = ipc // _NS
    nch = ips // chunk

    @functools.partial(
        pl.kernel,
        out_type=jax.ShapeDtypeStruct((_NC * n_rows, 16), F32),
        mesh=_sc_mesh(),
        compiler_params=_SC_PARAMS,
        scratch_types=[
            pltpu.VMEM((chunk,), I32),
            pltpu.VMEM((chunk, 16), F32),
            pltpu.VMEM_SHARED((n_rows, 16), F32),
        ],
    )
    def k(ones_hbm, idx_hbm, zeros_hbm, out_hbm, iv, vv, sh):
        cid = lax.axis_index("c")
        sid = lax.axis_index("s")

        @pl.when(sid == 0)
        def _():
            pltpu.sync_copy(zeros_hbm, sh)

        pltpu.sync_copy(ones_hbm, vv)
        plsc.subcore_barrier()
        base0 = cid * ipc + sid * ips
        for c in range(nch):
            b = base0 + c * chunk
            pltpu.sync_copy(idx_hbm.at[pl.ds(b, chunk)], iv)
            pltpu.sync_copy(vv, sh.at[iv], add=True)
        plsc.subcore_barrier()

        @pl.when(sid == 0)
        def _():
            pltpu.sync_copy(sh, out_hbm.at[pl.ds(cid * n_rows, n_rows)])

    return k(ones_c16, idx, zeros_16).reshape(_NC, n_rows, 16)


def _tc_prologue(v, w1t, w2t, cntp):
    n = v.shape[0]

    def body(v_ref, w1_ref, w2_ref, cntp_ref, a1_ref, a2_ref, cnt_ref):
        vv = v_ref[...]
        a1_ref[...] = jnp.dot(vv, w1_ref[...], preferred_element_type=F32)
        a2_ref[...] = jnp.dot(vv, w2_ref[...], preferred_element_type=F32)
        cp = cntp_ref[...]
        cnt_ref[...] = cp[0] + cp[1]

    return pl.pallas_call(
        body,
        out_shape=(
            jax.ShapeDtypeStruct((n, 16), F32),
            jax.ShapeDtypeStruct((n, 16), F32),
            jax.ShapeDtypeStruct((n, 16), F32),
        ),
    )(v, w1t, w2t, cntp)


def _tc_edge_mlp(prev_p, uprev_p, g1p, g2p, wa, wb, wc, wd, biases, ab, first):
    er = g1p.shape[0]
    blk = _EDGE_BLK
    grid = (er // blk,)
    row = pl.BlockSpec((blk, 128), lambda i: (i, 0))
    full = lambda s: pl.BlockSpec(s, lambda i: (0, 0))

    def body(*refs):
        if first:
            (prev_ref, g1_ref, g2_ref, wa_ref, wb_ref, wc_ref, wd_ref,
             b_ref, e_ref, u_ref, st_ref) = refs
        else:
            (prev_ref, up_ref, g1_ref, g2_ref, wb_ref, wc_ref, wd_ref,
             b_ref, ab_ref, e_ref, u_ref, st_ref) = refs
        i = pl.program_id(0)
        if first:
            e = jnp.dot(prev_ref[...], wa_ref[...], preferred_element_type=F32) \
                + b_ref[0:1, :]
        else:
            e = prev_ref[...] + up_ref[...] * ab_ref[0:1, :] + ab_ref[1:2, :]
        h = _lrelu(g1_ref[...] + g2_ref[...]
                   + jnp.dot(e, wb_ref[...], preferred_element_type=F32)
                   + b_ref[1:2, :])
        h = _lrelu(jnp.dot(h, wc_ref[...], preferred_element_type=F32)
                   + b_ref[2:3, :])
        u = jnp.dot(h, wd_ref[...], preferred_element_type=F32) + b_ref[3:4, :]
        e_ref[...] = e
        u_ref[...] = u

        @pl.when(i == 0)
        def _():
            st_ref[...] = jnp.zeros((8, 128), F32)

        st_ref[0:1, :] += jnp.sum(u, axis=0, keepdims=True)
        st_ref[1:2, :] += jnp.sum(u * u, axis=0, keepdims=True)

    if first:
        ins = (prev_p, g1p, g2p, wa, wb, wc, wd, biases)
        in_specs = [row, row, row, full((128, 128)), full((128, 128)),
                    full((128, 128)), full((128, 128)), full((4, 128))]
    else:
        ins = (prev_p, uprev_p, g1p, g2p, wb, wc, wd, biases, ab)
        in_specs = [row, row, row, row, full((128, 128)), full((128, 128)),
                    full((128, 128)), full((4, 128)), full((8, 128))]

    return pl.pallas_call(
        body,
        grid=grid,
        in_specs=in_specs,
        out_specs=(row, row, full((8, 128))),
        out_shape=(
            jax.ShapeDtypeStruct((er, 128), F32),
            jax.ShapeDtypeStruct((er, 128), F32),
            jax.ShapeDtypeStruct((8, 128), F32),
        ),
    )(*ins)


def _edge_affine(st, bn_g, bn_b, e_edges, fold, fold_t):
    su = jnp.dot(st[0:1, :], fold, preferred_element_type=F32, precision=HI)
    ssq = jnp.dot(st[1:2, :], fold, preferred_element_type=F32, precision=HI)
    mean = su / e_edges
    var = jnp.maximum(ssq / e_edges - mean * mean, 0.0)
    alpha = bn_g / jnp.sqrt(var + 1e-5)
    beta = bn_b - mean * alpha
    alpha_t = jnp.dot(alpha, fold_t, preferred_element_type=F32, precision=HI)
    beta_t = jnp.dot(beta, fold_t, preferred_element_type=F32, precision=HI)
    return alpha, beta, alpha_t, beta_t


def _node_mlp(vi, vv, wv1a_ref, wv1b_ref, wv2_ref, wv3_ref, bv_ref,
              bn2g_ref, bn2b_ref):
    y = _lrelu(jnp.dot(vi, wv1a_ref[...], preferred_element_type=F32)
               + jnp.dot(vv, wv1b_ref[...], preferred_element_type=F32)
               + bv_ref[0:1, :])
    y = _lrelu(jnp.dot(y, wv2_ref[...], preferred_element_type=F32)
               + bv_ref[1:2, :])
    y = jnp.dot(y, wv3_ref[...], preferred_element_type=F32) + bv_ref[2:3, :]
    ym = jnp.mean(y, axis=0, keepdims=True)
    yv = jnp.maximum(jnp.mean(y * y, axis=0, keepdims=True) - ym * ym, 0.0)
    an = bn2g_ref[...] / jnp.sqrt(yv + 1e-5)
    bn = bn2b_ref[...] - ym * an
    return vv + y * an + bn


def _tc_node_update(v, sp, cnt, st, fold, fold_t, bn1_g, bn1_b, wv1a, wv1b,
                    wv2, wv3, bv, bn2_g, bn2_b, w1n, w2n, e_edges):
    n = v.shape[0]

    def body(v_ref, sp_ref, cnt_ref, st_ref, fold_ref, foldt_ref, bn1g_ref,
             bn1b_ref, wv1a_ref, wv1b_ref, wv2_ref, wv3_ref, bv_ref, bn2g_ref,
             bn2b_ref, w1n_ref, w2n_ref, vn_ref, a1_ref, a2_ref, ab_ref):
        alpha, beta, alpha_t, beta_t = _edge_affine(
            st_ref[...], bn1g_ref[...], bn1b_ref[...], e_edges,
            fold_ref[...], foldt_ref[...])
        spv = sp_ref[...]
        s = spv[0] + spv[1]
        cntv = cnt_ref[...]
        vi = (s * alpha + cntv * beta) / jnp.maximum(cntv, 1.0)
        vn = _node_mlp(vi, v_ref[...], wv1a_ref, wv1b_ref, wv2_ref, wv3_ref,
                       bv_ref, bn2g_ref, bn2b_ref)
        vn_ref[...] = vn
        a1_ref[...] = jnp.dot(vn, w1n_ref[...], preferred_element_type=F32)
        a2_ref[...] = jnp.dot(vn, w2n_ref[...], preferred_element_type=F32)
        ab_ref[...] = jnp.concatenate(
            [alpha_t, beta_t, jnp.zeros((6, 128), F32)], axis=0)

    return pl.pallas_call(
        body,
        out_shape=(
            jax.ShapeDtypeStruct((n, 128), F32),
            jax.ShapeDtypeStruct((n, 16), F32),
            jax.ShapeDtypeStruct((n, 16), F32),
            jax.ShapeDtypeStruct((8, 128), F32),
        ),
    )(v, sp, cnt, st, fold, fold_t, bn1_g, bn1_b, wv1a, wv1b, wv2, wv3, bv,
      bn2_g, bn2_b, w1n, w2n)


def _tc_node_final(v, sp, sep, cnt, st, fold, fold_t, bn1_g, bn1_b, wv1a,
                   wv1b, wv2, wv3, bv, bn2_g, bn2_b, e_edges, n_pool):
    n = v.shape[0]

    def body(v_ref, sp_ref, sep_ref, cnt_ref, st_ref, fold_ref, foldt_ref,
             bn1g_ref, bn1b_ref, wv1a_ref, wv1b_ref, wv2_ref, wv3_ref, bv_ref,
             bn2g_ref, bn2b_ref, vn_ref, vif_ref):
        alpha, beta, _, _ = _edge_affine(
            st_ref[...], bn1g_ref[...], bn1b_ref[...], e_edges,
            fold_ref[...], foldt_ref[...])
        spv = sp_ref[...]
        sev = sep_ref[...]
        s = spv[0] + spv[1]
        se = sev[0] + sev[1]
        cntv = cnt_ref[...]
        denom = jnp.maximum(cntv, 1.0)
        vi = (s * alpha + cntv * beta) / denom
        vn = _node_mlp(vi, v_ref[...], wv1a_ref, wv1b_ref, wv2_ref, wv3_ref,
                       bv_ref, bn2g_ref, bn2b_ref)
        vi_fin = (se + s * alpha + cntv * beta) / denom
        pad = n_pool - n
        vn_ref[...] = jnp.concatenate(
            [vn, jnp.zeros((pad, 128), F32)], axis=0)
        vif_ref[...] = jnp.concatenate(
            [vi_fin, jnp.zeros((pad, 16), F32)], axis=0)

    return pl.pallas_call(
        body,
        out_shape=(
            jax.ShapeDtypeStruct((n_pool, 128), F32),
            jax.ShapeDtypeStruct((n_pool, 16), F32),
        ),
    )(v, sp, sep, cnt, st, fold, fold_t, bn1_g, bn1_b, wv1a, wv1b, wv2, wv3,
      bv, bn2_g, bn2_b)


def _tc_head(p16p, p128p, cnt3p, wca, wcb, bc, wf, bf, wo, bo, n_graphs):
    def body(p16_ref, p128_ref, c3_ref, wca_ref, wcb_ref, bc_ref, wf_ref,
             bf_ref, wo_ref, bo_ref, out_ref):
        g = n_graphs
        p16 = p16_ref[...]
        p128 = p128_ref[...]
        c3 = c3_ref[...]
        s16 = (p16[0] + p16[1])[:g]
        s128 = (p128[0] + p128[1])[:g]
        cnt3 = (c3[0] + c3[1])[:g, 0:1]
        d3 = jnp.maximum(cnt3, 1.0)
        m16 = s16 / d3
        m128 = s128 / d3
        h = _lrelu(jnp.dot(m16, wca_ref[...], preferred_element_type=F32)
                   + jnp.dot(m128, wcb_ref[...], preferred_element_type=F32)
                   + bc_ref[...])
        h = _lrelu(jnp.dot(h, wf_ref[...], preferred_element_type=F32)
                   + bf_ref[...])
        out_ref[...] = jnp.dot(h, wo_ref[...], preferred_element_type=F32) \
            + bo_ref[...]

    return pl.pallas_call(
        body,
        out_shape=jax.ShapeDtypeStruct((n_graphs, 128), F32),
    )(p16p, p128p, cnt3p, wca, wcb, bc, wf, bf, wo, bo)


def kernel(node_fea, edge_fea, idx1, idx2, idx3, params):
    n = node_fea.shape[0]
    e_edges = idx1.shape[0]
    n_graphs = 64
    er = e_edges * 16 // 128

    convs = params["convs"]
    fold = jnp.tile(jnp.eye(16, dtype=F32), (8, 1))
    fold_t = fold.T

    n_pad = ((n + 8 * _NW - 1) // (8 * _NW)) * (8 * _NW)
    nf_pad = jnp.pad(node_fea, (0, n_pad - n))
    v = _sc_emb_gather(params["v_emb"], nf_pad, n_pad, 128)[:n]

    zeros_n16 = jnp.zeros((n, 16), F32)
    ones_c16 = jnp.ones((_EDGE_CHUNK, 16), F32)

    cntp = _sc_seg_count(ones_c16, idx1, zeros_n16, n, e_edges, _EDGE_CHUNK)

    n_pool = n_pad
    pool_rows = 72
    idx3_pad = jnp.pad(idx3, (0, n_pool - n), constant_values=n_graphs)
    pool_chunk = n_pool // _NW
    zeros_p16 = jnp.zeros((pool_rows, 16), F32)
    zeros_p128 = jnp.zeros((pool_rows, 128), F32)
    ones_p16 = jnp.ones((pool_chunk, 16), F32)
    cnt3p = _sc_seg_count(ones_p16, idx3_pad, zeros_p16, pool_rows, n_pool,
                          pool_chunk)

    w_e1 = convs[0]["phi_e"][0]["W"]
    a1, a2, cnt = _tc_prologue(v, w_e1[:, :128].T, w_e1[:, 128:256].T,
                               cntp.reshape(_NC, n, 16))

    e_p = edge_fea.reshape(er, 128)
    u_p = None
    ab = jnp.zeros((8, 128), F32)

    out = None
    for k in range(3):
        cp = convs[k]
        w1 = cp["phi_e"][0]["W"]

        g1, g2 = _sc_edge_gather(a1, a2, idx1, idx2, e_edges)
        g1p = g1.reshape(er, 128)
        g2p = g2.reshape(er, 128)

        first = k == 0
        wa = _bd(params["e_emb"]["W"])
        biases = jnp.concatenate(
            [_tile8(params["e_emb"]["b"]), _tile8(cp["phi_e"][0]["b"]),
             _tile8(cp["phi_e"][1]["b"]), _tile8(cp["phi_e"][2]["b"])], axis=0)
        e_p, u_p, st = _tc_edge_mlp(
            e_p, u_p, g1p, g2p, wa, _bd(w1[:, 256:272]),
            _bd(cp["phi_e"][1]["W"]), _bd(cp["phi_e"][2]["W"]),
            biases, ab, first)

        u_rows = u_p.reshape(e_edges, 16)
        if k < 2:
            (sp,) = _sc_seg_sum([u_rows], [16], idx1, {16: zeros_n16}, n,
                                e_edges, _EDGE_CHUNK)
        else:
            e_rows = e_p.reshape(e_edges, 16)
            sp, se_p = _sc_seg_sum([u_rows, e_rows], [16, 16], idx1,
                                   {16: zeros_n16}, n, e_edges, _EDGE_CHUNK)

        wv1 = cp["phi_v"][0]["W"]
        bv = jnp.concatenate(
            [cp["phi_v"][0]["b"][None, :], cp["phi_v"][1]["b"][None, :],
             cp["phi_v"][2]["b"][None, :]], axis=0)
        common = dict(
            fold=fold, fold_t=fold_t, bn1_g=cp["bn1_g"][None, :],
            bn1_b=cp["bn1_b"][None, :], wv1a=wv1[:, :16].T, wv1b=wv1[:, 16:].T,
            wv2=cp["phi_v"][1]["W"].T, wv3=cp["phi_v"][2]["W"].T, bv=bv,
            bn2_g=cp["bn2_g"][None, :], bn2_b=cp["bn2_b"][None, :],
        )
        if k < 2:
            wn = convs[k + 1]["phi_e"][0]["W"]
            v, a1, a2, ab = _tc_node_update(
                v, sp, cnt, st, w1n=wn[:, :128].T, w2n=wn[:, 128:256].T,
                e_edges=e_edges, **common)
        else:
            vn_pool, vif_pool = _tc_node_final(
                v, sp, se_p, cnt, st, e_edges=e_edges, n_pool=n_pool,
                **common)
            p16p, p128p = _sc_seg_sum(
                [vif_pool, vn_pool], [16, 128], idx3_pad,
                {16: zeros_p16, 128: zeros_p128}, pool_rows, n_pool,
                pool_chunk)
            wc = params["conv_to_fc"]["W"]
            wo = jnp.pad(params["fc_out"]["W"].T, ((0, 0), (0, 126)))
            bo = jnp.pad(params["fc_out"]["b"][None, :], ((0, 0), (0, 126)))
            out = _tc_head(
                p16p, p128p, cnt3p, wca=wc[:, :16].T, wcb=wc[:, 16:].T,
                bc=params["conv_to_fc"]["b"][None, :],
                wf=params["fcs"][0]["W"].T, bf=params["fcs"][0]["b"][None, :],
                wo=wo, bo=bo, n_graphs=n_graphs)

    return out[:, :2]


# speedup vs baseline: 14.0803x; 1.0687x over previous
import functools

import jax
import jax.numpy as jnp
from jax import lax
from jax.experimental import pallas as pl
from jax.experimental.pallas import tpu as pltpu
from jax.experimental.pallas import tpu_sc as plsc

F32 = jnp.float32
I32 = jnp.int32
HI = jax.lax.Precision.HIGHEST

_NC = 2
_NS = 16
_NW = _NC * _NS

_EDGE_CHUNK = 2000
_EDGE_BLK = 1600

_SC_PARAMS = pltpu.CompilerParams(use_tc_tiling_on_sc=False)


def _sc_mesh():
    return plsc.VectorSubcoreMesh(core_axis_name="c", subcore_axis_name="s",
                                  num_cores=_NC, num_subcores=_NS)


def _wid():
    return lax.axis_index("s") * _NC + lax.axis_index("c")


def _lrelu(x):
    return jnp.where(x >= 0, x, 0.2 * x)


def _bd(w):
    return jnp.kron(jnp.eye(8, dtype=F32), w.T)


def _tile8(b):
    return jnp.tile(b, 8)[None, :]


def _sc_emb_gather(table, idx, n_pad, d):
    rpw = n_pad // _NW

    @functools.partial(
        pl.kernel,
        out_type=jax.ShapeDtypeStruct((n_pad, d), F32),
        mesh=_sc_mesh(),
        compiler_params=_SC_PARAMS,
        scratch_types=[
            pltpu.VMEM((rpw,), I32),
            pltpu.VMEM((rpw, d), F32),
            pltpu.SemaphoreType.DMA,
        ],
    )
    def k(table_hbm, idx_hbm, out_hbm, idx_v, rows_v, sem):
        base = _wid() * rpw
        pltpu.sync_copy(idx_hbm.at[pl.ds(base, rpw)], idx_v)
        pltpu.async_copy(table_hbm.at[idx_v], rows_v, sem).wait()
        pltpu.sync_copy(rows_v, out_hbm.at[pl.ds(base, rpw)])

    return k(table, idx)


def _sc_edge_gather(a1, a2, idx1, idx2, e_edges):
    chunk = _EDGE_CHUNK
    epw = e_edges // _NW
    nch = epw // chunk

    @functools.partial(
        pl.kernel,
        out_type=(
            jax.ShapeDtypeStruct((e_edges, 16), F32),
            jax.ShapeDtypeStruct((e_edges, 16), F32),
        ),
        mesh=_sc_mesh(),
        compiler_params=_SC_PARAMS,
        scratch_types=[
            pltpu.VMEM((chunk,), I32),
            pltpu.VMEM((chunk,), I32),
            pltpu.VMEM((chunk, 16), F32),
            pltpu.VMEM((chunk, 16), F32),
            pltpu.SemaphoreType.DMA,
        ],
    )
    def k(a1_hbm, a2_hbm, i1_hbm, i2_hbm, g1_hbm, g2_hbm, i1v, i2v, r1v, r2v, sem):
        base0 = _wid() * epw
        for c in range(nch):
            b = base0 + c * chunk
            pltpu.sync_copy(i1_hbm.at[pl.ds(b, chunk)], i1v)
            pltpu.sync_copy(i2_hbm.at[pl.ds(b, chunk)], i2v)
            d1 = pltpu.async_copy(a1_hbm.at[i1v], r1v, sem)
            d2 = pltpu.async_copy(a2_hbm.at[i2v], r2v, sem)
            d1.wait()
            d2.wait()
            pltpu.sync_copy(r1v, g1_hbm.at[pl.ds(b, chunk)])
            pltpu.sync_copy(r2v, g2_hbm.at[pl.ds(b, chunk)])

    return k(a1, a2, idx1, idx2)


def _sc_seg_sum(vals, widths, idx, zeros_map, n_rows, n_items, chunk):
    nv = len(vals)
    ipc = n_items // _NC
    ips = ipc // _NS
    nch = ips // chunk

    out_type = tuple(
        jax.ShapeDtypeStruct((_NC * n_rows, w), F32) for w in widths
    )
    scratch = (
        [pltpu.VMEM((chunk,), I32)]
        + [pltpu.VMEM((chunk, w), F32) for w in widths]
        + [pltpu.VMEM_SHARED((n_rows, w), F32) for w in widths]
    )
    zero_keys = sorted(zeros_map)
    zeros_in = [zeros_map[w] for w in zero_keys]
    nz = len(zeros_in)

    @functools.partial(
        pl.kernel,
        out_type=out_type if nv > 1 else out_type[0],
        mesh=_sc_mesh(),
        compiler_params=_SC_PARAMS,
        scratch_types=scratch,
    )
    def k(*refs):
        idx_hbm = refs[0]
        zero_hbms = {w: refs[1 + i] for i, w in enumerate(zero_keys)}
        val_hbms = refs[1 + nz : 1 + nz + nv]
        out_hbms = refs[1 + nz + nv : 1 + nz + 2 * nv]
        iv = refs[1 + nz + 2 * nv]
        vvs = refs[2 + nz + 2 * nv : 2 + nz + 3 * nv]
        shs = refs[2 + nz + 3 * nv : 2 + nz + 4 * nv]

        cid = lax.axis_index("c")
        sid = lax.axis_index("s")

        @pl.when(sid == 0)
        def _():
            for w, sh in zip(widths, shs):
                pltpu.sync_copy(zero_hbms[w], sh)

        plsc.subcore_barrier()
        base0 = cid * ipc + sid * ips
        for c in range(nch):
            b = base0 + c * chunk
            pltpu.sync_copy(idx_hbm.at[pl.ds(b, chunk)], iv)
            for vhbm, vv, sh in zip(val_hbms, vvs, shs):
                pltpu.sync_copy(vhbm.at[pl.ds(b, chunk)], vv)
                pltpu.sync_copy(vv, sh.at[iv], add=True)
        plsc.subcore_barrier()

        @pl.when(sid == 0)
        def _():
            for sh, ohbm in zip(shs, out_hbms):
                pltpu.sync_copy(sh, ohbm.at[pl.ds(cid * n_rows, n_rows)])

    outs = k(idx, *zeros_in, *vals)
    outs = outs if nv > 1 else (outs,)
    return [o.reshape(_NC, n_rows, w) for o, w in zip(outs, widths)]


def _sc_seg_count(ones_c16, idx, zeros_16, n_rows, n_items, chunk):
    ipc = n_items // _NC
    ips = ipc // _NS
    nch = ips // chunk

    @functools.partial(
        pl.kernel,
        out_type=jax.ShapeDtypeStruct((_NC * n_rows, 16), F32),
        mesh=_sc_mesh(),
        compiler_params=_SC_PARAMS,
        scratch_types=[
            pltpu.VMEM((chunk,), I32),
            pltpu.VMEM((chunk, 16), F32),
            pltpu.VMEM_SHARED((n_rows, 16), F32),
        ],
    )
    def k(ones_hbm, idx_hbm, zeros_hbm, out_hbm, iv, vv, sh):
        cid = lax.axis_index("c")
        sid = lax.axis_index("s")

        @pl.when(sid == 0)
        def _():
            pltpu.sync_copy(zeros_hbm, sh)

        pltpu.sync_copy(ones_hbm, vv)
        plsc.subcore_barrier()
        base0 = cid * ipc + sid * ips
        for c in range(nch):
            b = base0 + c * chunk
            pltpu.sync_copy(idx_hbm.at[pl.ds(b, chunk)], iv)
            pltpu.sync_copy(vv, sh.at[iv], add=True)
        plsc.subcore_barrier()

        @pl.when(sid == 0)
        def _():
            pltpu.sync_copy(sh, out_hbm.at[pl.ds(cid * n_rows, n_rows)])

    return k(ones_c16, idx, zeros_16).reshape(_NC, n_rows, 16)


def _tc_prologue(v, w1t, w2t, cntp):
    n = v.shape[0]

    def body(v_ref, w1_ref, w2_ref, cntp_ref, a1_ref, a2_ref, cnt_ref):
        vv = v_ref[...]
        a1_ref[...] = jnp.dot(vv, w1_ref[...], preferred_element_type=F32)
        a2_ref[...] = jnp.dot(vv, w2_ref[...], preferred_element_type=F32)
        cp = cntp_ref[...]
        cnt_ref[...] = cp[0] + cp[1]

    return pl.pallas_call(
        body,
        out_shape=(
            jax.ShapeDtypeStruct((n, 16), F32),
            jax.ShapeDtypeStruct((n, 16), F32),
            jax.ShapeDtypeStruct((n, 16), F32),
        ),
    )(v, w1t, w2t, cntp)


def _tc_edge_mlp(prev_p, uprev_p, g1p, g2p, wa, wb, wc, wd, biases, ab, first):
    er = g1p.shape[0]
    blk = _EDGE_BLK
    grid = (er // blk,)
    row = pl.BlockSpec((blk, 128), lambda i: (i, 0))
    full = lambda s: pl.BlockSpec(s, lambda i: (0, 0))

    def body(*refs):
        if first:
            (prev_ref, g1_ref, g2_ref, wa_ref, wb_ref, wc_ref, wd_ref,
             b_ref, e_ref, u_ref, st_ref) = refs
        else:
            (prev_ref, up_ref, g1_ref, g2_ref, wb_ref, wc_ref, wd_ref,
             b_ref, ab_ref, e_ref, u_ref, st_ref) = refs
        i = pl.program_id(0)
        if first:
            e = jnp.dot(prev_ref[...], wa_ref[...], preferred_element_type=F32) \
                + b_ref[0:1, :]
        else:
            e = prev_ref[...] + up_ref[...] * ab_ref[0:1, :] + ab_ref[1:2, :]
        h = _lrelu(g1_ref[...] + g2_ref[...]
                   + jnp.dot(e, wb_ref[...], preferred_element_type=F32)
                   + b_ref[1:2, :])
        h = _lrelu(jnp.dot(h, wc_ref[...], preferred_element_type=F32)
                   + b_ref[2:3, :])
        u = jnp.dot(h, wd_ref[...], preferred_element_type=F32) + b_ref[3:4, :]
        e_ref[...] = e
        u_ref[...] = u

        @pl.when(i == 0)
        def _():
            st_ref[...] = jnp.zeros((8, 128), F32)

        st_ref[0:1, :] += jnp.sum(u, axis=0, keepdims=True)
        st_ref[1:2, :] += jnp.sum(u * u, axis=0, keepdims=True)

    if first:
        ins = (prev_p, g1p, g2p, wa, wb, wc, wd, biases)
        in_specs = [row, row, row, full((128, 128)), full((128, 128)),
                    full((128, 128)), full((128, 128)), full((4, 128))]
    else:
        ins = (prev_p, uprev_p, g1p, g2p, wb, wc, wd, biases, ab)
        in_specs = [row, row, row, row, full((128, 128)), full((128, 128)),
                    full((128, 128)), full((4, 128)), full((8, 128))]

    return pl.pallas_call(
        body,
        grid=grid,
        in_specs=in_specs,
        out_specs=(row, row, full((8, 128))),
        out_shape=(
            jax.ShapeDtypeStruct((er, 128), F32),
            jax.ShapeDtypeStruct((er, 128), F32),
            jax.ShapeDtypeStruct((8, 128), F32),
        ),
    )(*ins)


def _edge_affine(st, bn_g, bn_b, e_edges, fold, fold_t):
    su = jnp.dot(st[0:1, :], fold, preferred_element_type=F32, precision=HI)
    ssq = jnp.dot(st[1:2, :], fold, preferred_element_type=F32, precision=HI)
    mean = su / e_edges
    var = jnp.maximum(ssq / e_edges - mean * mean, 0.0)
    alpha = bn_g / jnp.sqrt(var + 1e-5)
    beta = bn_b - mean * alpha
    alpha_t = jnp.dot(alpha, fold_t, preferred_element_type=F32, precision=HI)
    beta_t = jnp.dot(beta, fold_t, preferred_element_type=F32, precision=HI)
    return alpha, beta, alpha_t, beta_t


def _node_mlp(vi, vv, wv1a_ref, wv1b_ref, wv2_ref, wv3_ref, bv_ref,
              bn2g_ref, bn2b_ref):
    y = _lrelu(jnp.dot(vi, wv1a_ref[...], preferred_element_type=F32)
               + jnp.dot(vv, wv1b_ref[...], preferred_element_type=F32)
               + bv_ref[0:1, :])
    y = _lrelu(jnp.dot(y, wv2_ref[...], preferred_element_type=F32)
               + bv_ref[1:2, :])
    y = jnp.dot(y, wv3_ref[...], preferred_element_type=F32) + bv_ref[2:3, :]
    ym = jnp.mean(y, axis=0, keepdims=True)
    yv = jnp.maximum(jnp.mean(y * y, axis=0, keepdims=True) - ym * ym, 0.0)
    an = bn2g_ref[...] / jnp.sqrt(yv + 1e-5)
    bn = bn2b_ref[...] - ym * an
    return vv + y * an + bn


def _tc_node_update(v, sp, cnt, st, fold, fold_t, bn1_g, bn1_b, wv1a, wv1b,
                    wv2, wv3, bv, bn2_g, bn2_b, w1n, w2n, e_edges):
    n = v.shape[0]

    def body(v_ref, sp_ref, cnt_ref, st_ref, fold_ref, foldt_ref, bn1g_ref,
             bn1b_ref, wv1a_ref, wv1b_ref, wv2_ref, wv3_ref, bv_ref, bn2g_ref,
             bn2b_ref, w1n_ref, w2n_ref, vn_ref, a1_ref, a2_ref, ab_ref):
        alpha, beta, alpha_t, beta_t = _edge_affine(
            st_ref[...], bn1g_ref[...], bn1b_ref[...], e_edges,
            fold_ref[...], foldt_ref[...])
        spv = sp_ref[...]
        s = spv[0] + spv[1]
        cntv = cnt_ref[...]
        vi = (s * alpha + cntv * beta) / jnp.maximum(cntv, 1.0)
        vn = _node_mlp(vi, v_ref[...], wv1a_ref, wv1b_ref, wv2_ref, wv3_ref,
                       bv_ref, bn2g_ref, bn2b_ref)
        vn_ref[...] = vn
        a1_ref[...] = jnp.dot(vn, w1n_ref[...], preferred_element_type=F32)
        a2_ref[...] = jnp.dot(vn, w2n_ref[...], preferred_element_type=F32)
        ab_ref[...] = jnp.concatenate(
            [alpha_t, beta_t, jnp.zeros((6, 128), F32)], axis=0)

    return pl.pallas_call(
        body,
        out_shape=(
            jax.ShapeDtypeStruct((n, 128), F32),
            jax.ShapeDtypeStruct((n, 16), F32),
            jax.ShapeDtypeStruct((n, 16), F32),
            jax.ShapeDtypeStruct((8, 128), F32),
        ),
    )(v, sp, cnt, st, fold, fold_t, bn1_g, bn1_b, wv1a, wv1b, wv2, wv3, bv,
      bn2_g, bn2_b, w1n, w2n)


def _tc_node_final(v, sp, sep, cnt, st, fold, fold_t, bn1_g, bn1_b, wv1a,
                   wv1b, wv2, wv3, bv, bn2_g, bn2_b, e_edges, n_pool):
    n = v.shape[0]

    def body(v_ref, sp_ref, sep_ref, cnt_ref, st_ref, fold_ref, foldt_ref,
             bn1g_ref, bn1b_ref, wv1a_ref, wv1b_ref, wv2_ref, wv3_ref, bv_ref,
             bn2g_ref, bn2b_ref, vn_ref, vif_ref):
        alpha, beta, _, _ = _edge_affine(
            st_ref[...], bn1g_ref[...], bn1b_ref[...], e_edges,
            fold_ref[...], foldt_ref[...])
        spv = sp_ref[...]
        sev = sep_ref[...]
        s = spv[0] + spv[1]
        se = sev[0] + sev[1]
        cntv = cnt_ref[...]
        denom = jnp.maximum(cntv, 1.0)
        vi = (s * alpha + cntv * beta) / denom
        vn = _node_mlp(vi, v_ref[...], wv1a_ref, wv1b_ref, wv2_ref, wv3_ref,
                       bv_ref, bn2g_ref, bn2b_ref)
        vi_fin = (se + s * alpha + cntv * beta) / denom
        pad = n_pool - n
        vn_ref[...] = jnp.concatenate(
            [vn, jnp.zeros((pad, 128), F32)], axis=0)
        vif_ref[...] = jnp.concatenate(
            [vi_fin, jnp.zeros((pad, 16), F32)], axis=0)

    return pl.pallas_call(
        body,
        out_shape=(
            jax.ShapeDtypeStruct((n_pool, 128), F32),
            jax.ShapeDtypeStruct((n_pool, 16), F32),
        ),
    )(v, sp, sep, cnt, st, fold, fold_t, bn1_g, bn1_b, wv1a, wv1b, wv2, wv3,
      bv, bn2_g, bn2_b)


def _tc_head(p16p, p128p, cnt3p, wca, wcb, bc, wf, bf, wo, bo, n_graphs):
    def body(p16_ref, p128_ref, c3_ref, wca_ref, wcb_ref, bc_ref, wf_ref,
             bf_ref, wo_ref, bo_ref, out_ref):
        g = n_graphs
        p16 = p16_ref[...]
        p128 = p128_ref[...]
        c3 = c3_ref[...]
        s16 = (p16[0] + p16[1])[:g]
        s128 = (p128[0] + p128[1])[:g]
        cnt3 = (c3[0] + c3[1])[:g, 0:1]
        d3 = jnp.maximum(cnt3, 1.0)
        m16 = s16 / d3
        m128 = s128 / d3
        h = _lrelu(jnp.dot(m16, wca_ref[...], preferred_element_type=F32)
                   + jnp.dot(m128, wcb_ref[...], preferred_element_type=F32)
                   + bc_ref[...])
        h = _lrelu(jnp.dot(h, wf_ref[...], preferred_element_type=F32)
                   + bf_ref[...])
        out_ref[...] = jnp.dot(h, wo_ref[...], preferred_element_type=F32) \
            + bo_ref[...]

    return pl.pallas_call(
        body,
        out_shape=jax.ShapeDtypeStruct((n_graphs, 128), F32),
    )(p16p, p128p, cnt3p, wca, wcb, bc, wf, bf, wo, bo)


def kernel(node_fea, edge_fea, idx1, idx2, idx3, params):
    n = node_fea.shape[0]
    e_edges = idx1.shape[0]
    n_graphs = 64
    er = e_edges * 16 // 128

    convs = params["convs"]
    fold = jnp.tile(jnp.eye(16, dtype=F32), (8, 1))
    fold_t = fold.T

    n_pad = ((n + 8 * _NW - 1) // (8 * _NW)) * (8 * _NW)
    nf_pad = jnp.pad(node_fea, (0, n_pad - n))
    v = _sc_emb_gather(params["v_emb"], nf_pad, n_pad, 128)[:n]

    zeros_n16 = jnp.zeros((n, 16), F32)
    ones_c16 = jnp.ones((_EDGE_CHUNK, 16), F32)

    cntp = _sc_seg_count(ones_c16, idx1, zeros_n16, n, e_edges, _EDGE_CHUNK)

    n_pool = n_pad
    pool_rows = 72
    idx3_pad = jnp.pad(idx3, (0, n_pool - n), constant_values=n_graphs)
    pool_chunk = n_pool // _NW
    zeros_p16 = jnp.zeros((pool_rows, 16), F32)
    zeros_p128 = jnp.zeros((pool_rows, 128), F32)
    ones_p16 = jnp.ones((pool_chunk, 16), F32)
    cnt3p = _sc_seg_count(ones_p16, idx3_pad, zeros_p16, pool_rows, n_pool,
                          pool_chunk)

    w_e1 = convs[0]["phi_e"][0]["W"]
    a1, a2, cnt = _tc_prologue(v, w_e1[:, :128].T, w_e1[:, 128:256].T,
                               cntp.reshape(_NC, n, 16))

    e_p = (edge_fea.T.reshape(16, er, 8).transpose(1, 2, 0).reshape(er, 128))
    u_p = None
    ab = jnp.zeros((8, 128), F32)

    out = None
    for k in range(3):
        cp = convs[k]
        w1 = cp["phi_e"][0]["W"]

        g1, g2 = _sc_edge_gather(a1, a2, idx1, idx2, e_edges)
        g1p = g1.reshape(er, 128)
        g2p = g2.reshape(er, 128)

        first = k == 0
        wa = _bd(params["e_emb"]["W"])
        biases = jnp.concatenate(
            [_tile8(params["e_emb"]["b"]), _tile8(cp["phi_e"][0]["b"]),
             _tile8(cp["phi_e"][1]["b"]), _tile8(cp["phi_e"][2]["b"])], axis=0)
        e_p, u_p, st = _tc_edge_mlp(
            e_p, u_p, g1p, g2p, wa, _bd(w1[:, 256:272]),
            _bd(cp["phi_e"][1]["W"]), _bd(cp["phi_e"][2]["W"]),
            biases, ab, first)

        u_rows = u_p.reshape(e_edges, 16)
        if k < 2:
            (sp,) = _sc_seg_sum([u_rows], [16], idx1, {16: zeros_n16}, n,
                                e_edges, _EDGE_CHUNK)
        else:
            e_rows = e_p.reshape(e_edges, 16)
            sp, se_p = _sc_seg_sum([u_rows, e_rows], [16, 16], idx1,
                                   {16: zeros_n16}, n, e_edges, _EDGE_CHUNK)

        wv1 = cp["phi_v"][0]["W"]
        bv = jnp.concatenate(
            [cp["phi_v"][0]["b"][None, :], cp["phi_v"][1]["b"][None, :],
             cp["phi_v"][2]["b"][None, :]], axis=0)
        common = dict(
            fold=fold, fold_t=fold_t, bn1_g=cp["bn1_g"][None, :],
            bn1_b=cp["bn1_b"][None, :], wv1a=wv1[:, :16].T, wv1b=wv1[:, 16:].T,
            wv2=cp["phi_v"][1]["W"].T, wv3=cp["phi_v"][2]["W"].T, bv=bv,
            bn2_g=cp["bn2_g"][None, :], bn2_b=cp["bn2_b"][None, :],
        )
        if k < 2:
            wn = convs[k + 1]["phi_e"][0]["W"]
            v, a1, a2, ab = _tc_node_update(
                v, sp, cnt, st, w1n=wn[:, :128].T, w2n=wn[:, 128:256].T,
                e_edges=e_edges, **common)
        else:
            vn_pool, vif_pool = _tc_node_final(
                v, sp, se_p, cnt, st, e_edges=e_edges, n_pool=n_pool,
                **common)
            p16p, p128p = _sc_seg_sum(
                [vif_pool, vn_pool], [16, 128], idx3_pad,
                {16: zeros_p16, 128: zeros_p128}, pool_rows, n_pool,
                pool_chunk)
            wc = params["conv_to_fc"]["W"]
            wo = jnp.pad(params["fc_out"]["W"].T, ((0, 0), (0, 126)))
            bo = jnp.pad(params["fc_out"]["b"][None, :], ((0, 0), (0, 126)))
            out = _tc_head(
                p16p, p128p, cnt3p, wca=wc[:, :16].T, wcb=wc[:, 16:].T,
                bc=params["conv_to_fc"]["b"][None, :],
                wf=params["fcs"][0]["W"].T, bf=params["fcs"][0]["b"][None, :],
                wo=wo, bo=bo, n_graphs=n_graphs)

    return out[:, :2]


# speedup vs baseline: 15.0983x; 1.0723x over previous
import functools

import jax
import jax.numpy as jnp
from jax import lax
from jax.experimental import pallas as pl
from jax.experimental.pallas import tpu as pltpu
from jax.experimental.pallas import tpu_sc as plsc

F32 = jnp.float32
I32 = jnp.int32
HI = jax.lax.Precision.HIGHEST

_NC = 2
_NS = 16
_NW = _NC * _NS

_EDGE_CHUNK = 2000
_EDGE_BLK = 1600

_SC_PARAMS = pltpu.CompilerParams(use_tc_tiling_on_sc=False)


def _sc_mesh():
    return plsc.VectorSubcoreMesh(core_axis_name="c", subcore_axis_name="s",
                                  num_cores=_NC, num_subcores=_NS)


def _wid():
    return lax.axis_index("s") * _NC + lax.axis_index("c")


def _lrelu(x):
    return jnp.where(x >= 0, x, 0.2 * x)


def _bd(w):
    return jnp.kron(jnp.eye(8, dtype=F32), w.T)


def _tile8(b):
    return jnp.tile(b, 8)[None, :]


def _sc_emb_gather(table, idx, n_pad, d):
    rpw = n_pad // _NW

    @functools.partial(
        pl.kernel,
        out_type=jax.ShapeDtypeStruct((n_pad, d), F32),
        mesh=_sc_mesh(),
        compiler_params=_SC_PARAMS,
        scratch_types=[
            pltpu.VMEM((rpw,), I32),
            pltpu.VMEM((rpw, d), F32),
            pltpu.SemaphoreType.DMA,
        ],
    )
    def k(table_hbm, idx_hbm, out_hbm, idx_v, rows_v, sem):
        base = _wid() * rpw
        pltpu.sync_copy(idx_hbm.at[pl.ds(base, rpw)], idx_v)
        pltpu.async_copy(table_hbm.at[idx_v], rows_v, sem).wait()
        pltpu.sync_copy(rows_v, out_hbm.at[pl.ds(base, rpw)])

    return k(table, idx)


def _sc_edge_gather(a1, a2, idx1, idx2, e_edges):
    chunk = 1000
    epw = e_edges // _NW
    nch = epw // chunk

    @functools.partial(
        pl.kernel,
        out_type=(
            jax.ShapeDtypeStruct((e_edges, 16), F32),
            jax.ShapeDtypeStruct((e_edges, 16), F32),
        ),
        mesh=_sc_mesh(),
        compiler_params=_SC_PARAMS,
        scratch_types=[
            pltpu.VMEM((epw,), I32),
            pltpu.VMEM((epw,), I32),
            pltpu.VMEM((chunk, 16), F32),
            pltpu.VMEM((chunk, 16), F32),
            pltpu.VMEM((chunk, 16), F32),
            pltpu.VMEM((chunk, 16), F32),
            pltpu.SemaphoreType.DMA,
            pltpu.SemaphoreType.DMA,
        ],
    )
    def k(a1_hbm, a2_hbm, i1_hbm, i2_hbm, g1_hbm, g2_hbm, i1v, i2v,
          r1a, r1b, r2a, r2b, semg, semw):
        base0 = _wid() * epw
        pltpu.sync_copy(i1_hbm.at[pl.ds(base0, epw)], i1v)
        pltpu.sync_copy(i2_hbm.at[pl.ds(base0, epw)], i2v)
        r1 = (r1a, r1b)
        r2 = (r2a, r2b)

        def start_gather(c):
            return (
                pltpu.async_copy(a1_hbm.at[i1v.at[pl.ds(c * chunk, chunk)]],
                                 r1[c % 2], semg),
                pltpu.async_copy(a2_hbm.at[i2v.at[pl.ds(c * chunk, chunk)]],
                                 r2[c % 2], semg),
            )

        gd = {0: start_gather(0)}
        wd = {}
        for c in range(nch):
            gd[c][0].wait()
            gd[c][1].wait()
            b = base0 + c * chunk
            wd[c] = (
                pltpu.async_copy(r1[c % 2], g1_hbm.at[pl.ds(b, chunk)], semw),
                pltpu.async_copy(r2[c % 2], g2_hbm.at[pl.ds(b, chunk)], semw),
            )
            if c + 1 < nch:
                if c >= 1:
                    wd[c - 1][0].wait()
                    wd[c - 1][1].wait()
                gd[c + 1] = start_gather(c + 1)
        for c in (nch - 2, nch - 1):
            wd[c][0].wait()
            wd[c][1].wait()

    return k(a1, a2, idx1, idx2)


def _sc_seg_sum(vals, widths, idx, zeros_map, n_rows, n_items, chunk):
    nv = len(vals)
    ipc = n_items // _NC
    ips = ipc // _NS
    nch = ips // chunk

    out_type = tuple(
        jax.ShapeDtypeStruct((_NC * n_rows, w), F32) for w in widths
    )
    scratch = (
        [pltpu.VMEM((nch, chunk), I32)]
        + [pltpu.VMEM((chunk, w), F32) for w in widths for _ in range(2)]
        + [pltpu.VMEM_SHARED((n_rows, w), F32) for w in widths]
        + [pltpu.SemaphoreType.DMA, pltpu.SemaphoreType.DMA]
    )
    zero_keys = sorted(zeros_map)
    zeros_in = [zeros_map[w] for w in zero_keys]
    nz = len(zeros_in)

    @functools.partial(
        pl.kernel,
        out_type=out_type if nv > 1 else out_type[0],
        mesh=_sc_mesh(),
        compiler_params=_SC_PARAMS,
        scratch_types=scratch,
    )
    def k(*refs):
        idx_hbm = refs[0]
        zero_hbms = {w: refs[1 + i] for i, w in enumerate(zero_keys)}
        val_hbms = refs[1 + nz : 1 + nz + nv]
        out_hbms = refs[1 + nz + nv : 1 + nz + 2 * nv]
        iv = refs[1 + nz + 2 * nv]
        vbufs = refs[2 + nz + 2 * nv : 2 + nz + 2 * nv + 2 * nv]
        shs = refs[2 + nz + 4 * nv : 2 + nz + 5 * nv]
        semi, semv = refs[2 + nz + 5 * nv], refs[3 + nz + 5 * nv]

        cid = lax.axis_index("c")
        sid = lax.axis_index("s")
        base0 = cid * ipc + sid * ips

        idl = [pltpu.async_copy(idx_hbm.at[pl.ds(base0 + c * chunk, chunk)],
                                iv.at[c], semi) for c in range(nch)]

        def start_vals(c):
            return [pltpu.async_copy(
                vhbm.at[pl.ds(base0 + c * chunk, chunk)],
                vbufs[2 * j + c % 2], semv)
                for j, vhbm in enumerate(val_hbms)]

        vd = {0: start_vals(0)}

        @pl.when(sid == 0)
        def _():
            for w, sh in zip(widths, shs):
                pltpu.sync_copy(zero_hbms[w], sh)

        plsc.subcore_barrier()
        for c in range(nch):
            if c + 1 < nch:
                vd[c + 1] = start_vals(c + 1)
            for d in vd[c]:
                d.wait()
            idl[c].wait()
            for j, sh in enumerate(shs):
                pltpu.sync_copy(vbufs[2 * j + c % 2], sh.at[iv.at[c]],
                                add=True)
        plsc.subcore_barrier()

        @pl.when(sid == 0)
        def _():
            for sh, ohbm in zip(shs, out_hbms):
                pltpu.sync_copy(sh, ohbm.at[pl.ds(cid * n_rows, n_rows)])

    outs = k(idx, *zeros_in, *vals)
    outs = outs if nv > 1 else (outs,)
    return [o.reshape(_NC, n_rows, w) for o, w in zip(outs, widths)]


def _sc_seg_count(ones_c16, idx, zeros_16, n_rows, n_items, chunk):
    ipc = n_items // _NC
    ips = ipc // _NS
    nch = ips // chunk

    @functools.partial(
        pl.kernel,
        out_type=jax.ShapeDtypeStruct((_NC * n_rows, 16), F32),
        mesh=_sc_mesh(),
        compiler_params=_SC_PARAMS,
        scratch_types=[
            pltpu.VMEM((chunk,), I32),
            pltpu.VMEM((chunk, 16), F32),
            pltpu.VMEM_SHARED((n_rows, 16), F32),
        ],
    )
    def k(ones_hbm, idx_hbm, zeros_hbm, out_hbm, iv, vv, sh):
        cid = lax.axis_index("c")
        sid = lax.axis_index("s")

        @pl.when(sid == 0)
        def _():
            pltpu.sync_copy(zeros_hbm, sh)

        pltpu.sync_copy(ones_hbm, vv)
        plsc.subcore_barrier()
        base0 = cid * ipc + sid * ips
        for c in range(nch):
            b = base0 + c * chunk
            pltpu.sync_copy(idx_hbm.at[pl.ds(b, chunk)], iv)
            pltpu.sync_copy(vv, sh.at[iv], add=True)
        plsc.subcore_barrier()

        @pl.when(sid == 0)
        def _():
            pltpu.sync_copy(sh, out_hbm.at[pl.ds(cid * n_rows, n_rows)])

    return k(ones_c16, idx, zeros_16).reshape(_NC, n_rows, 16)


def _tc_prologue(v, w1t, w2t, cntp):
    n = v.shape[0]

    def body(v_ref, w1_ref, w2_ref, cntp_ref, a1_ref, a2_ref, cnt_ref):
        vv = v_ref[...]
        a1_ref[...] = jnp.dot(vv, w1_ref[...], preferred_element_type=F32)
        a2_ref[...] = jnp.dot(vv, w2_ref[...], preferred_element_type=F32)
        cp = cntp_ref[...]
        cnt_ref[...] = cp[0] + cp[1]

    return pl.pallas_call(
        body,
        out_shape=(
            jax.ShapeDtypeStruct((n, 16), F32),
            jax.ShapeDtypeStruct((n, 16), F32),
            jax.ShapeDtypeStruct((n, 16), F32),
        ),
    )(v, w1t, w2t, cntp)


def _tc_edge_mlp(prev_p, uprev_p, g1p, g2p, wa, wb, wc, wd, biases, ab, first):
    er = g1p.shape[0]
    blk = _EDGE_BLK
    grid = (er // blk,)
    row = pl.BlockSpec((blk, 128), lambda i: (i, 0))
    full = lambda s: pl.BlockSpec(s, lambda i: (0, 0))

    def body(*refs):
        if first:
            (prev_ref, g1_ref, g2_ref, wa_ref, wb_ref, wc_ref, wd_ref,
             b_ref, e_ref, u_ref, st_ref) = refs
        else:
            (prev_ref, up_ref, g1_ref, g2_ref, wb_ref, wc_ref, wd_ref,
             b_ref, ab_ref, e_ref, u_ref, st_ref) = refs
        i = pl.program_id(0)
        if first:
            e = jnp.dot(prev_ref[...], wa_ref[...], preferred_element_type=F32) \
                + b_ref[0:1, :]
        else:
            e = prev_ref[...] + up_ref[...] * ab_ref[0:1, :] + ab_ref[1:2, :]
        h = _lrelu(g1_ref[...] + g2_ref[...]
                   + jnp.dot(e, wb_ref[...], preferred_element_type=F32)
                   + b_ref[1:2, :])
        h = _lrelu(jnp.dot(h, wc_ref[...], preferred_element_type=F32)
                   + b_ref[2:3, :])
        u = jnp.dot(h, wd_ref[...], preferred_element_type=F32) + b_ref[3:4, :]
        e_ref[...] = e
        u_ref[...] = u

        @pl.when(i == 0)
        def _():
            st_ref[...] = jnp.zeros((8, 128), F32)

        st_ref[0:1, :] += jnp.sum(u, axis=0, keepdims=True)
        st_ref[1:2, :] += jnp.sum(u * u, axis=0, keepdims=True)

    if first:
        ins = (prev_p, g1p, g2p, wa, wb, wc, wd, biases)
        in_specs = [row, row, row, full((128, 128)), full((128, 128)),
                    full((128, 128)), full((128, 128)), full((4, 128))]
    else:
        ins = (prev_p, uprev_p, g1p, g2p, wb, wc, wd, biases, ab)
        in_specs = [row, row, row, row, full((128, 128)), full((128, 128)),
                    full((128, 128)), full((4, 128)), full((8, 128))]

    return pl.pallas_call(
        body,
        grid=grid,
        in_specs=in_specs,
        out_specs=(row, row, full((8, 128))),
        out_shape=(
            jax.ShapeDtypeStruct((er, 128), F32),
            jax.ShapeDtypeStruct((er, 128), F32),
            jax.ShapeDtypeStruct((8, 128), F32),
        ),
    )(*ins)


def _edge_affine(st, bn_g, bn_b, e_edges, fold, fold_t):
    su = jnp.dot(st[0:1, :], fold, preferred_element_type=F32, precision=HI)
    ssq = jnp.dot(st[1:2, :], fold, preferred_element_type=F32, precision=HI)
    mean = su / e_edges
    var = jnp.maximum(ssq / e_edges - mean * mean, 0.0)
    alpha = bn_g / jnp.sqrt(var + 1e-5)
    beta = bn_b - mean * alpha
    alpha_t = jnp.dot(alpha, fold_t, preferred_element_type=F32, precision=HI)
    beta_t = jnp.dot(beta, fold_t, preferred_element_type=F32, precision=HI)
    return alpha, beta, alpha_t, beta_t


def _node_mlp(vi, vv, wv1a_ref, wv1b_ref, wv2_ref, wv3_ref, bv_ref,
              bn2g_ref, bn2b_ref):
    y = _lrelu(jnp.dot(vi, wv1a_ref[...], preferred_element_type=F32)
               + jnp.dot(vv, wv1b_ref[...], preferred_element_type=F32)
               + bv_ref[0:1, :])
    y = _lrelu(jnp.dot(y, wv2_ref[...], preferred_element_type=F32)
               + bv_ref[1:2, :])
    y = jnp.dot(y, wv3_ref[...], preferred_element_type=F32) + bv_ref[2:3, :]
    ym = jnp.mean(y, axis=0, keepdims=True)
    yv = jnp.maximum(jnp.mean(y * y, axis=0, keepdims=True) - ym * ym, 0.0)
    an = bn2g_ref[...] / jnp.sqrt(yv + 1e-5)
    bn = bn2b_ref[...] - ym * an
    return vv + y * an + bn


def _tc_node_update(v, sp, cnt, st, fold, fold_t, bn1_g, bn1_b, wv1a, wv1b,
                    wv2, wv3, bv, bn2_g, bn2_b, w1n, w2n, e_edges):
    n = v.shape[0]

    def body(v_ref, sp_ref, cnt_ref, st_ref, fold_ref, foldt_ref, bn1g_ref,
             bn1b_ref, wv1a_ref, wv1b_ref, wv2_ref, wv3_ref, bv_ref, bn2g_ref,
             bn2b_ref, w1n_ref, w2n_ref, vn_ref, a1_ref, a2_ref, ab_ref):
        alpha, beta, alpha_t, beta_t = _edge_affine(
            st_ref[...], bn1g_ref[...], bn1b_ref[...], e_edges,
            fold_ref[...], foldt_ref[...])
        spv = sp_ref[...]
        s = spv[0] + spv[1]
        cntv = cnt_ref[...]
        vi = (s * alpha + cntv * beta) / jnp.maximum(cntv, 1.0)
        vn = _node_mlp(vi, v_ref[...], wv1a_ref, wv1b_ref, wv2_ref, wv3_ref,
                       bv_ref, bn2g_ref, bn2b_ref)
        vn_ref[...] = vn
        a1_ref[...] = jnp.dot(vn, w1n_ref[...], preferred_element_type=F32)
        a2_ref[...] = jnp.dot(vn, w2n_ref[...], preferred_element_type=F32)
        ab_ref[...] = jnp.concatenate(
            [alpha_t, beta_t, jnp.zeros((6, 128), F32)], axis=0)

    return pl.pallas_call(
        body,
        out_shape=(
            jax.ShapeDtypeStruct((n, 128), F32),
            jax.ShapeDtypeStruct((n, 16), F32),
            jax.ShapeDtypeStruct((n, 16), F32),
            jax.ShapeDtypeStruct((8, 128), F32),
        ),
    )(v, sp, cnt, st, fold, fold_t, bn1_g, bn1_b, wv1a, wv1b, wv2, wv3, bv,
      bn2_g, bn2_b, w1n, w2n)


def _tc_node_final(v, sp, sep, cnt, st, fold, fold_t, bn1_g, bn1_b, wv1a,
                   wv1b, wv2, wv3, bv, bn2_g, bn2_b, e_edges, n_pool):
    n = v.shape[0]

    def body(v_ref, sp_ref, sep_ref, cnt_ref, st_ref, fold_ref, foldt_ref,
             bn1g_ref, bn1b_ref, wv1a_ref, wv1b_ref, wv2_ref, wv3_ref, bv_ref,
             bn2g_ref, bn2b_ref, vn_ref, vif_ref):
        alpha, beta, _, _ = _edge_affine(
            st_ref[...], bn1g_ref[...], bn1b_ref[...], e_edges,
            fold_ref[...], foldt_ref[...])
        spv = sp_ref[...]
        sev = sep_ref[...]
        s = spv[0] + spv[1]
        se = sev[0] + sev[1]
        cntv = cnt_ref[...]
        denom = jnp.maximum(cntv, 1.0)
        vi = (s * alpha + cntv * beta) / denom
        vn = _node_mlp(vi, v_ref[...], wv1a_ref, wv1b_ref, wv2_ref, wv3_ref,
                       bv_ref, bn2g_ref, bn2b_ref)
        vi_fin = (se + s * alpha + cntv * beta) / denom
        pad = n_pool - n
        vn_ref[...] = jnp.concatenate(
            [vn, jnp.zeros((pad, 128), F32)], axis=0)
        vif_ref[...] = jnp.concatenate(
            [vi_fin, jnp.zeros((pad, 16), F32)], axis=0)

    return pl.pallas_call(
        body,
        out_shape=(
            jax.ShapeDtypeStruct((n_pool, 128), F32),
            jax.ShapeDtypeStruct((n_pool, 16), F32),
        ),
    )(v, sp, sep, cnt, st, fold, fold_t, bn1_g, bn1_b, wv1a, wv1b, wv2, wv3,
      bv, bn2_g, bn2_b)


def _tc_head(p16p, p128p, cnt3p, wca, wcb, bc, wf, bf, wo, bo, n_graphs):
    def body(p16_ref, p128_ref, c3_ref, wca_ref, wcb_ref, bc_ref, wf_ref,
             bf_ref, wo_ref, bo_ref, out_ref):
        g = n_graphs
        p16 = p16_ref[...]
        p128 = p128_ref[...]
        c3 = c3_ref[...]
        s16 = (p16[0] + p16[1])[:g]
        s128 = (p128[0] + p128[1])[:g]
        cnt3 = (c3[0] + c3[1])[:g, 0:1]
        d3 = jnp.maximum(cnt3, 1.0)
        m16 = s16 / d3
        m128 = s128 / d3
        h = _lrelu(jnp.dot(m16, wca_ref[...], preferred_element_type=F32)
                   + jnp.dot(m128, wcb_ref[...], preferred_element_type=F32)
                   + bc_ref[...])
        h = _lrelu(jnp.dot(h, wf_ref[...], preferred_element_type=F32)
                   + bf_ref[...])
        out_ref[...] = jnp.dot(h, wo_ref[...], preferred_element_type=F32) \
            + bo_ref[...]

    return pl.pallas_call(
        body,
        out_shape=jax.ShapeDtypeStruct((n_graphs, 128), F32),
    )(p16p, p128p, cnt3p, wca, wcb, bc, wf, bf, wo, bo)


def kernel(node_fea, edge_fea, idx1, idx2, idx3, params):
    n = node_fea.shape[0]
    e_edges = idx1.shape[0]
    n_graphs = 64
    er = e_edges * 16 // 128

    convs = params["convs"]
    fold = jnp.tile(jnp.eye(16, dtype=F32), (8, 1))
    fold_t = fold.T

    n_pad = ((n + 8 * _NW - 1) // (8 * _NW)) * (8 * _NW)
    nf_pad = jnp.pad(node_fea, (0, n_pad - n))
    v = _sc_emb_gather(params["v_emb"], nf_pad, n_pad, 128)[:n]

    zeros_n16 = jnp.zeros((n, 16), F32)
    ones_c16 = jnp.ones((_EDGE_CHUNK, 16), F32)

    cntp = _sc_seg_count(ones_c16, idx1, zeros_n16, n, e_edges, _EDGE_CHUNK)

    n_pool = n_pad
    pool_rows = 72
    idx3_pad = jnp.pad(idx3, (0, n_pool - n), constant_values=n_graphs)
    pool_chunk = n_pool // _NW
    zeros_p16 = jnp.zeros((pool_rows, 16), F32)
    zeros_p128 = jnp.zeros((pool_rows, 128), F32)
    ones_p16 = jnp.ones((pool_chunk, 16), F32)
    cnt3p = _sc_seg_count(ones_p16, idx3_pad, zeros_p16, pool_rows, n_pool,
                          pool_chunk)

    w_e1 = convs[0]["phi_e"][0]["W"]
    a1, a2, cnt = _tc_prologue(v, w_e1[:, :128].T, w_e1[:, 128:256].T,
                               cntp.reshape(_NC, n, 16))

    e_p = (edge_fea.T.reshape(16, er, 8).transpose(1, 2, 0).reshape(er, 128))
    u_p = None
    ab = jnp.zeros((8, 128), F32)

    out = None
    for k in range(3):
        cp = convs[k]
        w1 = cp["phi_e"][0]["W"]

        g1, g2 = _sc_edge_gather(a1, a2, idx1, idx2, e_edges)
        g1p = g1.reshape(er, 128)
        g2p = g2.reshape(er, 128)

        first = k == 0
        wa = _bd(params["e_emb"]["W"])
        biases = jnp.concatenate(
            [_tile8(params["e_emb"]["b"]), _tile8(cp["phi_e"][0]["b"]),
             _tile8(cp["phi_e"][1]["b"]), _tile8(cp["phi_e"][2]["b"])], axis=0)
        e_p, u_p, st = _tc_edge_mlp(
            e_p, u_p, g1p, g2p, wa, _bd(w1[:, 256:272]),
            _bd(cp["phi_e"][1]["W"]), _bd(cp["phi_e"][2]["W"]),
            biases, ab, first)

        u_rows = u_p.reshape(e_edges, 16)
        if k < 2:
            (sp,) = _sc_seg_sum([u_rows], [16], idx1, {16: zeros_n16}, n,
                                e_edges, 1000)
        else:
            e_rows = e_p.reshape(e_edges, 16)
            sp, se_p = _sc_seg_sum([u_rows, e_rows], [16, 16], idx1,
                                   {16: zeros_n16}, n, e_edges, 1000)

        wv1 = cp["phi_v"][0]["W"]
        bv = jnp.concatenate(
            [cp["phi_v"][0]["b"][None, :], cp["phi_v"][1]["b"][None, :],
             cp["phi_v"][2]["b"][None, :]], axis=0)
        common = dict(
            fold=fold, fold_t=fold_t, bn1_g=cp["bn1_g"][None, :],
            bn1_b=cp["bn1_b"][None, :], wv1a=wv1[:, :16].T, wv1b=wv1[:, 16:].T,
            wv2=cp["phi_v"][1]["W"].T, wv3=cp["phi_v"][2]["W"].T, bv=bv,
            bn2_g=cp["bn2_g"][None, :], bn2_b=cp["bn2_b"][None, :],
        )
        if k < 2:
            wn = convs[k + 1]["phi_e"][0]["W"]
            v, a1, a2, ab = _tc_node_update(
                v, sp, cnt, st, w1n=wn[:, :128].T, w2n=wn[:, 128:256].T,
                e_edges=e_edges, **common)
        else:
            vn_pool, vif_pool = _tc_node_final(
                v, sp, se_p, cnt, st, e_edges=e_edges, n_pool=n_pool,
                **common)
            p16p, p128p = _sc_seg_sum(
                [vif_pool, vn_pool], [16, 128], idx3_pad,
                {16: zeros_p16, 128: zeros_p128}, pool_rows, n_pool,
                pool_chunk)
            wc = params["conv_to_fc"]["W"]
            wo = jnp.pad(params["fc_out"]["W"].T, ((0, 0), (0, 126)))
            bo = jnp.pad(params["fc_out"]["b"][None, :], ((0, 0), (0, 126)))
            out = _tc_head(
                p16p, p128p, cnt3p, wca=wc[:, :16].T, wcb=wc[:, 16:].T,
                bc=params["conv_to_fc"]["b"][None, :],
                wf=params["fcs"][0]["W"].T, bf=params["fcs"][0]["b"][None, :],
                wo=wo, bo=bo, n_graphs=n_graphs)

    return out[:, :2]
